# Initial kernel scaffold; baseline (speedup 1.0000x reference)
import jax
import jax.numpy as jnp
from jax.experimental import pallas as pl


def kernel(adj, x, W1, b1, W_mu, b_mu, W_lv, b_lv, eps):
    raise NotImplementedError("write your pallas kernel here")


# speedup vs baseline: 1.4358x; 1.4358x over previous
import jax
import jax.numpy as jnp
from jax.experimental import pallas as pl
from jax.experimental.pallas import tpu as pltpu


def _stage1_kernel(adj_ref, x_ref, w1_ref, b1_ref, wcat_ref, q_ref, p_ref):
    @pl.when(pl.program_id(0) == 0)
    def _():
        p_ref[...] = jnp.dot(x_ref[...], w1_ref[...],
                             preferred_element_type=jnp.float32)

    h = jnp.maximum(
        jnp.dot(adj_ref[...], p_ref[...], preferred_element_type=jnp.float32)
        + b1_ref[...], 0.0)
    q_ref[...] = jnp.dot(h, wcat_ref[...], preferred_element_type=jnp.float32)


def _stage2_kernel(adj_ref, q_ref, bcat_ref, eps_ref, z_ref):
    zd = z_ref.shape[-1]
    o = jnp.dot(adj_ref[...], q_ref[...], preferred_element_type=jnp.float32)
    o = o + bcat_ref[...]
    z_ref[...] = o[:, :zd] + jnp.exp(0.5 * o[:, zd:]) * eps_ref[...]


def kernel(adj, x, W1, b1, W_mu, b_mu, W_lv, b_lv, eps):
    n, _ = adj.shape
    xd = x.shape[1]
    hd = W1.shape[1]
    zd = W_mu.shape[1]

    rb = 400
    grid = (n // rb,)

    wcat = jnp.concatenate([W_mu, W_lv], axis=1)
    bcat = jnp.concatenate([b_mu, b_lv]).reshape(1, 2 * zd)
    b1r = b1.reshape(1, hd)

    q = pl.pallas_call(
        _stage1_kernel,
        grid=grid,
        in_specs=[
            pl.BlockSpec((rb, n), lambda i: (i, 0)),
            pl.BlockSpec((n, xd), lambda i: (0, 0)),
            pl.BlockSpec((xd, hd), lambda i: (0, 0)),
            pl.BlockSpec((1, hd), lambda i: (0, 0)),
            pl.BlockSpec((hd, 2 * zd), lambda i: (0, 0)),
        ],
        out_specs=pl.BlockSpec((rb, 2 * zd), lambda i: (i, 0)),
        out_shape=jax.ShapeDtypeStruct((n, 2 * zd), jnp.float32),
        scratch_shapes=[pltpu.VMEM((n, hd), jnp.float32)],
        compiler_params=pltpu.CompilerParams(
            dimension_semantics=("arbitrary",)),
    )(adj, x, W1, b1r, wcat)

    z = pl.pallas_call(
        _stage2_kernel,
        grid=grid,
        in_specs=[
            pl.BlockSpec((rb, n), lambda i: (i, 0)),
            pl.BlockSpec((n, 2 * zd), lambda i: (0, 0)),
            pl.BlockSpec((1, 2 * zd), lambda i: (0, 0)),
            pl.BlockSpec((rb, zd), lambda i: (i, 0)),
        ],
        out_specs=pl.BlockSpec((rb, zd), lambda i: (i, 0)),
        out_shape=jax.ShapeDtypeStruct((n, zd), jnp.float32),
        compiler_params=pltpu.CompilerParams(
            dimension_semantics=("parallel",)),
    )(adj, q, bcat, eps)

    return z
